# baseline (speedup 1.0000x reference)
import functools

import jax
import jax.numpy as jnp
from jax import lax
from jax.experimental import pallas as pl
from jax.experimental.pallas import tpu as pltpu
from jax.experimental.pallas import tpu_sc as plsc

N = 10000
E = 320000
D = 128

NC = 2
NS = 16
NW = NC * NS
CHUNK = 128
NTRASH = 240
EPW = E // NW
EPWP = EPW + NTRASH
NCHUNK = EPWP // CHUNK
NP = 10240
RPT = NP // NS
BD = 20
BA = 20
NBUF = 5

_MESH = plsc.VectorSubcoreMesh(core_axis_name="c", subcore_axis_name="s")


@functools.partial(
    pl.kernel,
    out_type=jax.ShapeDtypeStruct((NC * NP, 16), jnp.float32),
    mesh=_MESH,
    scratch_types=[
        pltpu.VMEM((BD, CHUNK), jnp.int32),
        pltpu.VMEM((CHUNK, 16), jnp.float32),
        pltpu.VMEM((RPT, 16), jnp.float32),
        pltpu.VMEM_SHARED((NP, 16), jnp.float32),
        pltpu.SemaphoreType.DMA,
    ],
    compiler_params=pltpu.CompilerParams(use_tc_tiling_on_sc=False),
)
def _sc_degree(dst_hbm, ones_hbm, zeros_hbm, out_hbm, didx_v, ones_v, stage_v,
               acc_sh, sem):
    cid = lax.axis_index("c")
    sid = lax.axis_index("s")
    wid = sid * NC + cid

    pltpu.sync_copy(ones_hbm, ones_v)

    pltpu.sync_copy(zeros_hbm, acc_sh.at[pl.ds(sid * RPT, RPT)])
    plsc.subcore_barrier()

    def blk(b, _):
        pltpu.sync_copy(dst_hbm.at[wid, b], didx_v)

        def fire(i, _):
            pltpu.async_copy(ones_v, acc_sh.at[didx_v.at[i]], sem, add=True)
            return ()

        def drain(i, _):
            pltpu.make_async_copy(ones_v, acc_sh.at[didx_v.at[i]], sem).wait()
            return ()

        lax.fori_loop(0, BD, fire, ())
        lax.fori_loop(0, BD, drain, ())
        return ()

    lax.fori_loop(0, NCHUNK // BD, blk, ())
    plsc.subcore_barrier()

    pltpu.sync_copy(acc_sh.at[pl.ds(sid * RPT, RPT)],
                    out_hbm.at[pl.ds(cid * NP + sid * RPT, RPT)])


HD = D // NC
EPS = E // NS
PAD_A = 480
EPSP = EPS + PAD_A
NCHUNK_A = EPSP // CHUNK


@functools.partial(
    pl.kernel,
    out_type=jax.ShapeDtypeStruct((NC * NP, HD), jnp.float32),
    mesh=_MESH,
    scratch_types=[
        pltpu.VMEM((BA, CHUNK), jnp.int32),
        pltpu.VMEM((BA, CHUNK), jnp.int32),
        pltpu.VMEM((BA, CHUNK), jnp.int32),
        pltpu.VMEM((BA, CHUNK), jnp.int32),
        [pltpu.VMEM((CHUNK, HD), jnp.float32) for _ in range(NBUF)],
        pltpu.VMEM((RPT, HD), jnp.float32),
        pltpu.VMEM_SHARED((NP, HD), jnp.float32),
        pltpu.SemaphoreType.DMA,
        pltpu.SemaphoreType.DMA,
        [pltpu.SemaphoreType.DMA for _ in range(NBUF)],
        [pltpu.SemaphoreType.DMA for _ in range(NBUF)],
    ],
    compiler_params=pltpu.CompilerParams(use_tc_tiling_on_sc=False),
)
def _sc_aggregate(src2_hbm, dst_hbm, xs2_hbm, zeros_hbm, out_hbm,
                  sidxa_v, didxa_v, sidxb_v, didxb_v, rows_v, stage_v, acc_sh,
                  isema, isemb, gsems, ssems):
    cid = lax.axis_index("c")
    sid = lax.axis_index("s")
    w = cid * NS + sid
    STEPS = BA // NBUF
    NBLK = NCHUNK_A // BA
    pairs = ((sidxa_v, didxa_v, isema), (sidxb_v, didxb_v, isemb))

    def stage(blki, p):
        sidx, didx, isem = p
        pltpu.async_copy(src2_hbm.at[w, blki], sidx, isem)
        pltpu.async_copy(dst_hbm.at[sid, blki], didx, isem)

    def wait_stage(blki, p):
        sidx, didx, isem = p
        pltpu.make_async_copy(src2_hbm.at[w, blki], sidx, isem).wait()
        pltpu.make_async_copy(dst_hbm.at[sid, blki], didx, isem).wait()

    def gather(p, i, b):
        return pltpu.async_copy(xs2_hbm.at[p[0].at[i]], rows_v[b], gsems[b])

    def wait_gather(p, i, b):
        pltpu.make_async_copy(xs2_hbm.at[p[0].at[i]], rows_v[b],
                              gsems[b]).wait()

    def scatter(p, i, b):
        return pltpu.async_copy(rows_v[b], acc_sh.at[p[1].at[i]], ssems[b],
                                add=True)

    def wait_scatter(p, i, b):
        pltpu.make_async_copy(rows_v[b], acc_sh.at[p[1].at[i]],
                              ssems[b]).wait()

    stage(0, pairs[0])
    pltpu.sync_copy(zeros_hbm, acc_sh.at[pl.ds(sid * RPT, RPT)])
    plsc.subcore_barrier()
    wait_stage(0, pairs[0])
    for b in range(NBUF):
        gather(pairs[0], b, b)

    def run_block(cur, nxt, pre_refill):
        def body(s, _):
            i0 = s * NBUF
            for b in range(NBUF):
                wait_gather(cur, i0 + b, b)
                scatter(cur, i0 + b, b)
            for b in range(NBUF):
                wait_scatter(cur, i0 + b, b)
                gather(cur, i0 + NBUF + b, b)
            return ()

        lax.fori_loop(0, STEPS - 1, body, ())
        i0 = (STEPS - 1) * NBUF
        for b in range(NBUF):
            wait_gather(cur, i0 + b, b)
            scatter(cur, i0 + b, b)
        pre_refill()
        for b in range(NBUF):
            wait_scatter(cur, i0 + b, b)
            gather(nxt, b, b)

    for blki in range(NBLK):
        cur = pairs[blki % 2]
        nxt = pairs[(blki + 1) % 2]
        if blki + 1 < NBLK:
            stage(blki + 1, nxt)
            run_block(cur, nxt, functools.partial(wait_stage, blki + 1, nxt))
        else:
            run_block(cur, cur, lambda: None)
    for b in range(NBUF):
        wait_gather(pairs[(NBLK - 1) % 2], b, b)
    plsc.subcore_barrier()

    pltpu.sync_copy(acc_sh.at[pl.ds(sid * RPT, RPT)],
                    out_hbm.at[pl.ds(cid * NP + sid * RPT, RPT)])


RB = 1000
GRID = N // RB


def _prep_body(degp_ref, x_ref, xs2_ref, dinv_ref):
    deg = degp_ref[0, :, 0:1] + degp_ref[1, :, 0:1] + 1.0
    dinv = lax.rsqrt(deg)
    dinv_ref[...] = dinv
    xs = x_ref[...] * dinv
    xs2_ref[0] = xs[:, :HD]
    xs2_ref[1] = xs[:, HD:]


def _tc_prep(degp, x):
    return pl.pallas_call(
        _prep_body,
        grid=(GRID,),
        in_specs=[
            pl.BlockSpec((2, RB, 16), lambda i: (0, i, 0)),
            pl.BlockSpec((RB, D), lambda i: (i, 0)),
        ],
        out_specs=[
            pl.BlockSpec((2, RB, HD), lambda i: (0, i, 0)),
            pl.BlockSpec((RB, 1), lambda i: (i, 0)),
        ],
        out_shape=[
            jax.ShapeDtypeStruct((2, N, HD), jnp.float32),
            jax.ShapeDtypeStruct((N, 1), jnp.float32),
        ],
    )(degp, x)


def _mid_body(g1_ref, xs2_ref, dinv_ref, w1_ref, b1_ref, w2_ref,
              ms2_ref):
    dinv = dinv_ref[...]
    z0 = (g1_ref[0] + xs2_ref[0]) * dinv
    z1 = (g1_ref[1] + xs2_ref[1]) * dinv
    h = (jnp.dot(z0, w1_ref[:HD], preferred_element_type=jnp.float32)
         + jnp.dot(z1, w1_ref[HD:], preferred_element_type=jnp.float32))
    h = jnp.maximum(h + b1_ref[...], 0.0)
    m = jnp.dot(h, w2_ref[...], preferred_element_type=jnp.float32)
    ms = m * dinv
    ms2_ref[0] = ms[:, :HD]
    ms2_ref[1] = ms[:, HD:]


def _tc_mid(g1p, xs2, dinv, w1p, b1p, w2p):
    hp = w1p.shape[1]
    return pl.pallas_call(
        _mid_body,
        grid=(GRID,),
        in_specs=[
            pl.BlockSpec((2, RB, HD), lambda i: (0, i, 0)),
            pl.BlockSpec((2, RB, HD), lambda i: (0, i, 0)),
            pl.BlockSpec((RB, 1), lambda i: (i, 0)),
            pl.BlockSpec((D, hp), lambda i: (0, 0)),
            pl.BlockSpec((1, hp), lambda i: (0, 0)),
            pl.BlockSpec((hp, D), lambda i: (0, 0)),
        ],
        out_specs=pl.BlockSpec((2, RB, HD), lambda i: (0, i, 0)),
        out_shape=jax.ShapeDtypeStruct((2, N, HD), jnp.float32),
    )(g1p, xs2, dinv, w1p, b1p, w2p)


def _final_body(g2_ref, ms2_ref, dinv_ref, b2_ref, out_ref):
    dinv = dinv_ref[...]
    out_ref[:, :HD] = (
        (g2_ref[0] + ms2_ref[0]) * dinv + b2_ref[:, :HD]
    )
    out_ref[:, HD:] = (
        (g2_ref[1] + ms2_ref[1]) * dinv + b2_ref[:, HD:]
    )


def _tc_final(g2p, ms2, dinv, b2):
    return pl.pallas_call(
        _final_body,
        grid=(GRID,),
        in_specs=[
            pl.BlockSpec((2, RB, HD), lambda i: (0, i, 0)),
            pl.BlockSpec((2, RB, HD), lambda i: (0, i, 0)),
            pl.BlockSpec((RB, 1), lambda i: (i, 0)),
            pl.BlockSpec((1, D), lambda i: (0, 0)),
        ],
        out_specs=pl.BlockSpec((RB, D), lambda i: (i, 0)),
        out_shape=jax.ShapeDtypeStruct((N, D), jnp.float32),
    )(g2p, ms2, dinv, b2)


HP = 384


def kernel(x, edge_index, W1, b1, W2, b2):
    src = edge_index[0]
    dst = edge_index[1]

    pad_s = jnp.zeros((NS, PAD_A), jnp.int32)
    pad_d = jnp.broadcast_to(
        N + (jnp.arange(PAD_A, dtype=jnp.int32) % NTRASH), (NS, PAD_A))
    srcp = jnp.concatenate([src.reshape(NS, EPS), pad_s], axis=1)
    dstp = jnp.concatenate([dst.reshape(NS, EPS), pad_d], axis=1)
    src2r = jnp.concatenate([srcp, srcp + N], axis=0).reshape(
        NW, NCHUNK_A // BA, BA, CHUNK)
    dstr_a = dstp.reshape(NS, NCHUNK_A // BA, BA, CHUNK)

    pad_dd = jnp.broadcast_to(
        N + (jnp.arange(NTRASH, dtype=jnp.int32) % NTRASH), (NW, NTRASH))
    dstd = jnp.concatenate([dst.reshape(NW, EPW), pad_dd], axis=1)
    dstr_d = dstd.reshape(NW, NCHUNK // BD, BD, CHUNK)
    ones16 = jnp.ones((CHUNK, 16), jnp.float32)

    zeros16 = jnp.zeros((RPT, 16), jnp.float32)
    zerosh = jnp.zeros((RPT, HD), jnp.float32)

    hidden = W1.shape[1]
    w1p = jnp.pad(W1, ((0, 0), (0, HP - hidden)))
    b1p = jnp.pad(b1, (0, HP - hidden)).reshape(1, HP)
    w2p = jnp.pad(W2, ((0, HP - hidden), (0, 0)))
    b2r = b2.reshape(1, D)

    degp = _sc_degree(dstr_d, ones16, zeros16)

    xs2, dinv = _tc_prep(degp.reshape(NC, NP, 16), x)

    g1p = _sc_aggregate(src2r, dstr_a, xs2.reshape(NC * N, HD), zerosh)
    ms2 = _tc_mid(g1p.reshape(NC, NP, HD), xs2, dinv, w1p, b1p, w2p)

    g2p = _sc_aggregate(src2r, dstr_a, ms2.reshape(NC * N, HD), zerosh)
    return _tc_final(g2p.reshape(NC, NP, HD), ms2, dinv, b2r)


# speedup vs baseline: 1.7788x; 1.7788x over previous
import functools

import jax
import jax.numpy as jnp
from jax import lax
from jax.experimental import pallas as pl
from jax.experimental.pallas import tpu as pltpu
from jax.experimental.pallas import tpu_sc as plsc

N = 10000
E = 320000
D = 128

NC = 2
NS = 16
NW = NC * NS
EPW = E // NW
CHUNK = 80
NCHUNK = EPW // CHUNK
NP = 10240
RPT = NP // NS
BD = 25
BA = 50
NBUF = 5

_MESH = plsc.VectorSubcoreMesh(core_axis_name="c", subcore_axis_name="s")


@functools.partial(
    pl.kernel,
    out_type=jax.ShapeDtypeStruct((NC * NP, 16), jnp.float32),
    mesh=_MESH,
    scratch_types=[
        pltpu.VMEM((BD, CHUNK), jnp.int32),
        pltpu.VMEM((CHUNK, 16), jnp.float32),
        pltpu.VMEM((RPT, 16), jnp.float32),
        pltpu.VMEM_SHARED((NP, 16), jnp.float32),
        pltpu.SemaphoreType.DMA,
    ],
    compiler_params=pltpu.CompilerParams(use_tc_tiling_on_sc=False),
)
def _sc_degree(dst_hbm, ones_hbm, zeros_hbm, out_hbm, didx_v, ones_v, stage_v,
               acc_sh, sem):
    cid = lax.axis_index("c")
    sid = lax.axis_index("s")
    wid = sid * NC + cid

    pltpu.sync_copy(ones_hbm, ones_v)

    pltpu.sync_copy(zeros_hbm, acc_sh.at[pl.ds(sid * RPT, RPT)])
    plsc.subcore_barrier()

    def blk(b, _):
        pltpu.sync_copy(dst_hbm.at[wid, b], didx_v)

        def fire(i, _):
            pltpu.async_copy(ones_v, acc_sh.at[didx_v.at[i]], sem, add=True)
            return ()

        def drain(i, _):
            pltpu.make_async_copy(ones_v, acc_sh.at[didx_v.at[i]], sem).wait()
            return ()

        lax.fori_loop(0, BD, fire, ())
        lax.fori_loop(0, BD, drain, ())
        return ()

    lax.fori_loop(0, NCHUNK // BD, blk, ())
    plsc.subcore_barrier()

    pltpu.sync_copy(acc_sh.at[pl.ds(sid * RPT, RPT)],
                    out_hbm.at[pl.ds(cid * NP + sid * RPT, RPT)])


HD = D // NC
EPS = E // NS
NCHUNK_A = EPS // CHUNK


@functools.partial(
    pl.kernel,
    out_type=jax.ShapeDtypeStruct((NC * NP, HD), jnp.float32),
    mesh=_MESH,
    scratch_types=[
        pltpu.VMEM((BA, CHUNK), jnp.int32),
        pltpu.VMEM((BA, CHUNK), jnp.int32),
        pltpu.VMEM((BA, CHUNK), jnp.int32),
        pltpu.VMEM((BA, CHUNK), jnp.int32),
        [pltpu.VMEM((CHUNK, HD), jnp.float32) for _ in range(NBUF)],
        pltpu.VMEM((RPT, HD), jnp.float32),
        pltpu.VMEM_SHARED((NP, HD), jnp.float32),
        pltpu.SemaphoreType.DMA,
        pltpu.SemaphoreType.DMA,
        [pltpu.SemaphoreType.DMA for _ in range(NBUF)],
        [pltpu.SemaphoreType.DMA for _ in range(NBUF)],
    ],
    compiler_params=pltpu.CompilerParams(use_tc_tiling_on_sc=False),
)
def _sc_aggregate(src2_hbm, dst_hbm, xs2_hbm, zeros_hbm, out_hbm,
                  sidxa_v, didxa_v, sidxb_v, didxb_v, rows_v, stage_v, acc_sh,
                  isema, isemb, gsems, ssems):
    cid = lax.axis_index("c")
    sid = lax.axis_index("s")
    w = cid * NS + sid
    STEPS = BA // NBUF
    NBLK = NCHUNK_A // BA
    pairs = ((sidxa_v, didxa_v, isema), (sidxb_v, didxb_v, isemb))

    def stage(blki, p):
        sidx, didx, isem = p
        pltpu.async_copy(src2_hbm.at[w, blki], sidx, isem)
        pltpu.async_copy(dst_hbm.at[sid, blki], didx, isem)

    def wait_stage(blki, p):
        sidx, didx, isem = p
        pltpu.make_async_copy(src2_hbm.at[w, blki], sidx, isem).wait()
        pltpu.make_async_copy(dst_hbm.at[sid, blki], didx, isem).wait()

    def gather(p, i, b):
        return pltpu.async_copy(xs2_hbm.at[p[0].at[i]], rows_v[b], gsems[b])

    def wait_gather(p, i, b):
        pltpu.make_async_copy(xs2_hbm.at[p[0].at[i]], rows_v[b],
                              gsems[b]).wait()

    def scatter(p, i, b):
        return pltpu.async_copy(rows_v[b], acc_sh.at[p[1].at[i]], ssems[b],
                                add=True)

    def wait_scatter(p, i, b):
        pltpu.make_async_copy(rows_v[b], acc_sh.at[p[1].at[i]],
                              ssems[b]).wait()

    stage(0, pairs[0])
    pltpu.sync_copy(zeros_hbm, acc_sh.at[pl.ds(sid * RPT, RPT)])
    plsc.subcore_barrier()
    wait_stage(0, pairs[0])
    for b in range(NBUF):
        gather(pairs[0], b, b)

    def run_block(cur, nxt, pre_refill):
        def body(s, _):
            i0 = s * NBUF
            for b in range(NBUF):
                wait_gather(cur, i0 + b, b)
                scatter(cur, i0 + b, b)
            for b in range(NBUF):
                wait_scatter(cur, i0 + b, b)
                gather(cur, i0 + NBUF + b, b)
            return ()

        lax.fori_loop(0, STEPS - 1, body, ())
        i0 = (STEPS - 1) * NBUF
        for b in range(NBUF):
            wait_gather(cur, i0 + b, b)
            scatter(cur, i0 + b, b)
        pre_refill()
        for b in range(NBUF):
            wait_scatter(cur, i0 + b, b)
            gather(nxt, b, b)

    for blki in range(NBLK):
        cur = pairs[blki % 2]
        nxt = pairs[(blki + 1) % 2]
        if blki + 1 < NBLK:
            stage(blki + 1, nxt)
            run_block(cur, nxt, functools.partial(wait_stage, blki + 1, nxt))
        else:
            run_block(cur, cur, lambda: None)
    for b in range(NBUF):
        wait_gather(pairs[(NBLK - 1) % 2], b, b)
    plsc.subcore_barrier()

    pltpu.sync_copy(acc_sh.at[pl.ds(sid * RPT, RPT)],
                    out_hbm.at[pl.ds(cid * NP + sid * RPT, RPT)])


RB = 1000
GRID = N // RB


def _prep_body(degp_ref, x_ref, xs2_ref, dinv_ref):
    deg = degp_ref[0, :, 0:1] + degp_ref[1, :, 0:1] + 1.0
    dinv = lax.rsqrt(deg)
    dinv_ref[...] = dinv
    xs = x_ref[...] * dinv
    xs2_ref[0] = xs[:, :HD]
    xs2_ref[1] = xs[:, HD:]


def _tc_prep(degp, x):
    return pl.pallas_call(
        _prep_body,
        grid=(GRID,),
        in_specs=[
            pl.BlockSpec((2, RB, 16), lambda i: (0, i, 0)),
            pl.BlockSpec((RB, D), lambda i: (i, 0)),
        ],
        out_specs=[
            pl.BlockSpec((2, RB, HD), lambda i: (0, i, 0)),
            pl.BlockSpec((RB, 1), lambda i: (i, 0)),
        ],
        out_shape=[
            jax.ShapeDtypeStruct((2, N, HD), jnp.float32),
            jax.ShapeDtypeStruct((N, 1), jnp.float32),
        ],
    )(degp, x)


def _mid_body(g1_ref, xs2_ref, dinv_ref, w1_ref, b1_ref, w2_ref,
              ms2_ref):
    dinv = dinv_ref[...]
    z0 = (g1_ref[0] + xs2_ref[0]) * dinv
    z1 = (g1_ref[1] + xs2_ref[1]) * dinv
    h = (jnp.dot(z0, w1_ref[:HD], preferred_element_type=jnp.float32)
         + jnp.dot(z1, w1_ref[HD:], preferred_element_type=jnp.float32))
    h = jnp.maximum(h + b1_ref[...], 0.0)
    m = jnp.dot(h, w2_ref[...], preferred_element_type=jnp.float32)
    ms = m * dinv
    ms2_ref[0] = ms[:, :HD]
    ms2_ref[1] = ms[:, HD:]


def _tc_mid(g1p, xs2, dinv, w1p, b1p, w2p):
    hp = w1p.shape[1]
    return pl.pallas_call(
        _mid_body,
        grid=(GRID,),
        in_specs=[
            pl.BlockSpec((2, RB, HD), lambda i: (0, i, 0)),
            pl.BlockSpec((2, RB, HD), lambda i: (0, i, 0)),
            pl.BlockSpec((RB, 1), lambda i: (i, 0)),
            pl.BlockSpec((D, hp), lambda i: (0, 0)),
            pl.BlockSpec((1, hp), lambda i: (0, 0)),
            pl.BlockSpec((hp, D), lambda i: (0, 0)),
        ],
        out_specs=pl.BlockSpec((2, RB, HD), lambda i: (0, i, 0)),
        out_shape=jax.ShapeDtypeStruct((2, N, HD), jnp.float32),
    )(g1p, xs2, dinv, w1p, b1p, w2p)


def _final_body(g2_ref, ms2_ref, dinv_ref, b2_ref, out_ref):
    dinv = dinv_ref[...]
    out_ref[:, :HD] = (
        (g2_ref[0] + ms2_ref[0]) * dinv + b2_ref[:, :HD]
    )
    out_ref[:, HD:] = (
        (g2_ref[1] + ms2_ref[1]) * dinv + b2_ref[:, HD:]
    )


def _tc_final(g2p, ms2, dinv, b2):
    return pl.pallas_call(
        _final_body,
        grid=(GRID,),
        in_specs=[
            pl.BlockSpec((2, RB, HD), lambda i: (0, i, 0)),
            pl.BlockSpec((2, RB, HD), lambda i: (0, i, 0)),
            pl.BlockSpec((RB, 1), lambda i: (i, 0)),
            pl.BlockSpec((1, D), lambda i: (0, 0)),
        ],
        out_specs=pl.BlockSpec((RB, D), lambda i: (i, 0)),
        out_shape=jax.ShapeDtypeStruct((N, D), jnp.float32),
    )(g2p, ms2, dinv, b2)


HP = 384


def kernel(x, edge_index, W1, b1, W2, b2):
    src = edge_index[0]
    dst = edge_index[1]
    src2r = jnp.concatenate([src, src + N]).reshape(
        NW, NCHUNK_A // BA, BA, CHUNK)
    dstr_a = dst.reshape(NS, NCHUNK_A // BA, BA, CHUNK)
    dstr_d = dst.reshape(NW, NCHUNK // BD, BD, CHUNK)
    ones16 = jnp.ones((CHUNK, 16), jnp.float32)

    zeros16 = jnp.zeros((RPT, 16), jnp.float32)
    zerosh = jnp.zeros((RPT, HD), jnp.float32)

    hidden = W1.shape[1]
    w1p = jnp.pad(W1, ((0, 0), (0, HP - hidden)))
    b1p = jnp.pad(b1, (0, HP - hidden)).reshape(1, HP)
    w2p = jnp.pad(W2, ((0, HP - hidden), (0, 0)))
    b2r = b2.reshape(1, D)

    degp = _sc_degree(dstr_d, ones16, zeros16)

    xs2, dinv = _tc_prep(degp.reshape(NC, NP, 16), x)

    g1p = _sc_aggregate(src2r, dstr_a, xs2.reshape(NC * N, HD), zerosh)
    ms2 = _tc_mid(g1p.reshape(NC, NP, HD), xs2, dinv, w1p, b1p, w2p)

    g2p = _sc_aggregate(src2r, dstr_a, ms2.reshape(NC * N, HD), zerosh)
    return _tc_final(g2p.reshape(NC, NP, HD), ms2, dinv, b2r)
